# baseline (speedup 1.0000x reference)
import jax
import jax.numpy as jnp
from jax.experimental import pallas as pl
from jax.experimental.pallas import tpu as pltpu

Q = 1024
D = 128
K = 100000
QBLK = 256
KBLK = 4096
CW = 512
NQ = Q // QBLK
NK = (K + KBLK - 1) // KBLK
KPAD = NK * KBLK
NCH = KBLK // CW
IMAX = 2**31 - 1


def _nn_kernel(q_ref, k_ref, min_ref, idx_ref, kn_ref, sm_ref, sa_ref):
    kid = pl.program_id(0)
    i = pl.program_id(1)

    @pl.when(i == 0)
    def _knorms():
        kw = k_ref[...]
        kn = jnp.sum(kw * kw, axis=1, keepdims=True)
        gk = (jax.lax.broadcasted_iota(jnp.int32, (KBLK, 1), 0)
              + kid * KBLK)
        kn = jnp.where(gk < K, kn, jnp.inf)
        kn_ref[...] = kn.reshape(NCH, CW)

    q = q_ref[...]
    qn = jnp.sum(q * q, axis=1, keepdims=True)
    q2 = q * -2.0
    coli = jax.lax.broadcasted_iota(jnp.int32, (QBLK, CW), 1)
    gbase = kid * KBLK

    def body(c, carry):
        best, arg = carry
        kc = k_ref[pl.ds(c * CW, CW), :]
        qk2 = jax.lax.dot_general(
            q2, kc, (((1,), (1,)), ((), ())),
            preferred_element_type=jnp.float32)
        kn_c = kn_ref[pl.ds(c, 1), :]
        e = (qn + kn_c) + qk2
        cand = coli + (gbase + c * CW)
        take = e < best
        best = jnp.minimum(best, e)
        arg = jnp.where(take, cand, arg)
        return best, arg

    best0 = jnp.full((QBLK, CW), jnp.inf, dtype=jnp.float32)
    arg0 = jnp.zeros((QBLK, CW), dtype=jnp.int32)
    best, arg = jax.lax.fori_loop(0, NCH, body, (best0, arg0))

    rowmin = jnp.min(best, axis=1, keepdims=True)
    rowarg = jnp.min(jnp.where(best == rowmin, arg, IMAX),
                     axis=1, keepdims=True)

    sl = pl.ds(i * QBLK, QBLK)

    @pl.when(kid == 0)
    def _init():
        sm_ref[sl, :] = rowmin
        sa_ref[sl, :] = rowarg

    @pl.when(kid > 0)
    def _merge():
        prev = sm_ref[sl, :]
        take = rowmin < prev
        sm_ref[sl, :] = jnp.where(take, rowmin, prev)
        sa_ref[sl, :] = jnp.where(take, rowarg, sa_ref[sl, :])

    @pl.when(kid == NK - 1)
    def _finish():
        min_ref[...] = jnp.sqrt(jnp.maximum(sm_ref[sl, :], 1e-12))
        idx_ref[...] = sa_ref[sl, :]


@jax.jit
def kernel(queries, keys):
    keys_p = jnp.pad(keys, ((0, KPAD - K), (0, 0)))
    min_d, idx = pl.pallas_call(
        _nn_kernel,
        grid=(NK, NQ),
        in_specs=[
            pl.BlockSpec((QBLK, D), lambda k, i: (i, 0)),
            pl.BlockSpec((KBLK, D), lambda k, i: (k, 0)),
        ],
        out_specs=[
            pl.BlockSpec((QBLK, 1), lambda k, i: (i, 0)),
            pl.BlockSpec((QBLK, 1), lambda k, i: (i, 0)),
        ],
        out_shape=[
            jax.ShapeDtypeStruct((Q, 1), jnp.float32),
            jax.ShapeDtypeStruct((Q, 1), jnp.int32),
        ],
        scratch_shapes=[
            pltpu.VMEM((NCH, CW), jnp.float32),
            pltpu.VMEM((Q, 1), jnp.float32),
            pltpu.VMEM((Q, 1), jnp.int32),
        ],
    )(queries, keys_p)
    return (min_d[:, 0], idx[:, 0])


# speedup vs baseline: 1.9295x; 1.9295x over previous
import jax
import jax.numpy as jnp
from jax.experimental import pallas as pl
from jax.experimental.pallas import tpu as pltpu

Q = 1024
D = 128
K = 100000
QBLK = 128
KBLK = 4096
CW = 256
NQ = Q // QBLK
NK = (K + KBLK - 1) // KBLK
KPAD = NK * KBLK
NCH = KBLK // CW
IMAX = 2**31 - 1
PADV = 1.0e6


def _nn_kernel(q_ref, k_ref, min_ref, idx_ref, kn_ref, sb_ref, sc_ref):
    kid = pl.program_id(0)
    i = pl.program_id(1)

    @pl.when(i == 0)
    def _knorms():
        kw = k_ref[...]
        kn = jnp.sum(kw * kw, axis=1, keepdims=True)
        kn_ref[...] = kn.reshape(NCH, CW)

    q = q_ref[...]
    qn = jnp.sum(q * q, axis=1, keepdims=True)
    q2 = q * -2.0
    sl = pl.ds(i * QBLK, QBLK)

    def window(best, argc):
        for c in range(NCH):
            kc = k_ref[pl.ds(c * CW, CW), :]
            qk2 = jax.lax.dot_general(
                q2, kc, (((1,), (1,)), ((), ())),
                preferred_element_type=jnp.float32)
            kn_c = kn_ref[pl.ds(c, 1), :]
            e = (qn + kn_c) + qk2
            take = e < best
            best = jnp.minimum(best, e)
            argc = jnp.where(take, kid * NCH + c, argc)
        return best, argc

    @pl.when(kid == 0)
    def _init():
        sb_ref[sl, :] = jnp.full((QBLK, CW), jnp.inf, dtype=jnp.float32)
        sc_ref[sl, :] = jnp.zeros((QBLK, CW), dtype=jnp.int32)

    best, argc = window(sb_ref[sl, :], sc_ref[sl, :])
    sb_ref[sl, :] = best
    sc_ref[sl, :] = argc

    @pl.when(kid == NK - 1)
    def _finish():
        best = sb_ref[sl, :]
        argc = sc_ref[sl, :]
        coli = jax.lax.broadcasted_iota(jnp.int32, (QBLK, CW), 1)
        gidx = argc * CW + coli
        rowmin = jnp.min(best, axis=1, keepdims=True)
        rowarg = jnp.min(jnp.where(best == rowmin, gidx, IMAX),
                         axis=1, keepdims=True)
        min_ref[...] = jnp.sqrt(jnp.maximum(rowmin, 1e-12))
        idx_ref[...] = rowarg


@jax.jit
def kernel(queries, keys):
    pad = jnp.zeros((KPAD - K, D), jnp.float32).at[:, 0].set(PADV)
    keys_p = jnp.concatenate([keys, pad], axis=0)
    min_d, idx = pl.pallas_call(
        _nn_kernel,
        grid=(NK, NQ),
        in_specs=[
            pl.BlockSpec((QBLK, D), lambda k, i: (i, 0)),
            pl.BlockSpec((KBLK, D), lambda k, i: (k, 0)),
        ],
        out_specs=[
            pl.BlockSpec((QBLK, 1), lambda k, i: (i, 0)),
            pl.BlockSpec((QBLK, 1), lambda k, i: (i, 0)),
        ],
        out_shape=[
            jax.ShapeDtypeStruct((Q, 1), jnp.float32),
            jax.ShapeDtypeStruct((Q, 1), jnp.int32),
        ],
        scratch_shapes=[
            pltpu.VMEM((NCH, CW), jnp.float32),
            pltpu.VMEM((Q, CW), jnp.float32),
            pltpu.VMEM((Q, CW), jnp.int32),
        ],
    )(queries, keys_p)
    return (min_d[:, 0], idx[:, 0])
